# Initial kernel scaffold; baseline (speedup 1.0000x reference)
import jax
import jax.numpy as jnp
from jax.experimental import pallas as pl


def kernel(user_ids, product_ids, category_ids, cluster_ids, behavior_scores, wide_user_W, wide_cat_W, user_W, product_W, cluster_W, W1, b1, W2, b2, W3, b3, W4, b4):
    raise NotImplementedError("write your pallas kernel here")


# speedup vs baseline: 2.9067x; 2.9067x over previous
import functools

import jax
import jax.numpy as jnp
import numpy as np
from jax import lax
from jax.experimental import pallas as pl
from jax.experimental.pallas import tpu as pltpu
from jax.experimental.pallas import tpu_sc as plsc

B = 16384
EMB = 128
NC = 2
NS = 16
NW = NC * NS
RPW = B // NW
SUB = 128
NSUB = RPW // SUB


def _sc_gather(u3, p3, c3, k3, user_W, product_W, cluster_W, wide_user_W, wide_cat_W):
    mesh = plsc.VectorSubcoreMesh(core_axis_name="c", subcore_axis_name="s")

    @functools.partial(
        pl.kernel,
        mesh=mesh,
        out_type=[
            jax.ShapeDtypeStruct((B, EMB), jnp.float32),
            jax.ShapeDtypeStruct((B, EMB), jnp.float32),
            jax.ShapeDtypeStruct((B, EMB), jnp.float32),
            jax.ShapeDtypeStruct((B, EMB), jnp.float32),
            jax.ShapeDtypeStruct((B, EMB), jnp.float32),
        ],
        scratch_types=[
            pltpu.VMEM((NSUB, SUB), jnp.int32),
            pltpu.VMEM((NSUB, SUB), jnp.int32),
            pltpu.VMEM((NSUB, SUB), jnp.int32),
            pltpu.VMEM((NSUB, SUB), jnp.int32),
            pltpu.VMEM((RPW, EMB), jnp.float32),
            pltpu.SemaphoreType.DMA,
        ],
    )
    def k(u_hbm, p_hbm, c_hbm, k_hbm, uW, pW, cW, wuW, wcW,
          pu_o, pi_o, pc_o, wu_o, wc_o,
          iu, ip, ic, ik, buf, sem):
        wid = lax.axis_index("s") * NC + lax.axis_index("c")
        base = wid * RPW
        pltpu.sync_copy(u_hbm.at[wid], iu)
        pltpu.sync_copy(p_hbm.at[wid], ip)
        pltpu.sync_copy(c_hbm.at[wid], ic)
        pltpu.sync_copy(k_hbm.at[wid], ik)
        for table, idx, out in (
            (uW, iu, pu_o),
            (pW, ip, pi_o),
            (wuW, iu, wu_o),
            (wcW, ic, wc_o),
            (cW, ik, pc_o),
        ):
            cps = [
                pltpu.async_copy(table.at[idx.at[j]], buf.at[pl.ds(j * SUB, SUB)], sem)
                for j in range(NSUB)
            ]
            for cp in cps:
                cp.wait()
            pltpu.sync_copy(buf, out.at[pl.ds(base, RPW)])

    return k(u3, p3, c3, k3, user_W, product_W, cluster_W, wide_user_W, wide_cat_W)


def _tc_body(pu_r, pi_r, pc_r, wu_r, wc_r, bs_r,
             A1_r, A2_r, A3_r, w1v_r, b1_r, A4_r, b2_r, A5_r, b3_r, A6_r, b4_r,
             o_r):
    f32 = jnp.float32
    acc = jnp.dot(pu_r[...], A1_r[...], preferred_element_type=f32)
    acc = acc + jnp.dot(pi_r[...], A2_r[...], preferred_element_type=f32)
    acc = acc + jnp.dot(pc_r[...], A3_r[...], preferred_element_type=f32)
    acc = acc + bs_r[...] * w1v_r[...] + b1_r[...]
    h = jnp.maximum(acc, 0.0)
    h = jnp.maximum(jnp.dot(h, A4_r[...], preferred_element_type=f32) + b2_r[...], 0.0)
    h = jnp.maximum(jnp.dot(h, A5_r[...], preferred_element_type=f32) + b3_r[...], 0.0)
    logit = jnp.dot(h, A6_r[...], preferred_element_type=f32) + b4_r[...]
    wide = jnp.sum(wu_r[...] * wc_r[...], axis=1, keepdims=True)
    o_r[...] = 1.0 / (1.0 + jnp.exp(-(logit + wide)))


def _tc_mlp(pu, pi, pc, wu, wc, bs2,
            A1, A2, A3, w1v, b1s, A4, b2s, A5, b3s, A6, b4r):
    bm = 512
    grid = (B // bm,)

    def blk(shape):
        return pl.BlockSpec(shape, lambda i: (i, 0))

    def full(a):
        return pl.BlockSpec(a.shape, lambda i: (0, 0))

    return pl.pallas_call(
        _tc_body,
        grid=grid,
        in_specs=[
            blk((bm, EMB)), blk((bm, EMB)), blk((bm, EMB)),
            blk((bm, EMB)), blk((bm, EMB)), blk((bm, 1)),
            full(A1), full(A2), full(A3), full(w1v), full(b1s),
            full(A4), full(b2s), full(A5), full(b3s), full(A6), full(b4r),
        ],
        out_specs=blk((bm, 1)),
        out_shape=jax.ShapeDtypeStruct((B, 1), jnp.float32),
    )(pu, pi, pc, wu, wc, bs2, A1, A2, A3, w1v, b1s, A4, b2s, A5, b3s, A6, b4r)


def kernel(user_ids, product_ids, category_ids, cluster_ids, behavior_scores,
           wide_user_W, wide_cat_W, user_W, product_W, cluster_W,
           W1, b1, W2, b2, W3, b3, W4, b4):
    u3 = user_ids.astype(jnp.int32).reshape(NW, NSUB, SUB)
    p3 = product_ids.astype(jnp.int32).reshape(NW, NSUB, SUB)
    c3 = category_ids.astype(jnp.int32).reshape(NW, NSUB, SUB)
    k3 = cluster_ids.astype(jnp.int32).reshape(NW, NSUB, SUB)

    cluster_Wp = jnp.pad(cluster_W, ((0, 0), (0, EMB // 2)))
    pu, pi, pc, wu, wc = _sc_gather(
        u3, p3, c3, k3, user_W, product_W, cluster_Wp, wide_user_W, wide_cat_W)

    s = float(1.0 / np.sqrt(1.0 + 1e-5))
    W1s = W1 * s
    A1 = W1s[:, :EMB].T
    A2 = W1s[:, EMB:2 * EMB].T
    A3 = jnp.pad(W1s[:, 2 * EMB:2 * EMB + EMB // 2].T, ((0, EMB // 2), (0, 0)))
    w1v = W1s[:, 2 * EMB + EMB // 2].reshape(1, -1)
    b1s = (b1 * s).reshape(1, -1)
    A4 = (W2 * s).T
    b2s = (b2 * s).reshape(1, -1)
    A5 = (W3 * s).T
    b3s = (b3 * s).reshape(1, -1)
    A6 = W4.T
    b4r = b4.reshape(1, 1)

    out2 = _tc_mlp(pu, pi, pc, wu, wc, behavior_scores.reshape(B, 1),
                   A1, A2, A3, w1v, b1s, A4, b2s, A5, b3s, A6, b4r)
    return out2.reshape(B)


# speedup vs baseline: 2.9129x; 1.0021x over previous
import functools

import jax
import jax.numpy as jnp
import numpy as np
from jax import lax
from jax.experimental import pallas as pl
from jax.experimental.pallas import tpu as pltpu
from jax.experimental.pallas import tpu_sc as plsc

B = 16384
EMB = 128
NC = 2
NS = 16
NW = NC * NS
RPW = B // NW
SUB = 128
NSUB = RPW // SUB


def _sc_gather(u3, p3, c3, k3, user_W, product_W, cluster_W, wide_user_W, wide_cat_W):
    mesh = plsc.VectorSubcoreMesh(core_axis_name="c", subcore_axis_name="s")

    @functools.partial(
        pl.kernel,
        mesh=mesh,
        out_type=[
            jax.ShapeDtypeStruct((B, EMB), jnp.float32),
            jax.ShapeDtypeStruct((B, EMB), jnp.float32),
            jax.ShapeDtypeStruct((B, EMB), jnp.float32),
            jax.ShapeDtypeStruct((B, EMB), jnp.float32),
            jax.ShapeDtypeStruct((B, EMB), jnp.float32),
        ],
        scratch_types=[
            pltpu.VMEM((NSUB, SUB), jnp.int32),
            pltpu.VMEM((NSUB, SUB), jnp.int32),
            pltpu.VMEM((NSUB, SUB), jnp.int32),
            pltpu.VMEM((NSUB, SUB), jnp.int32),
            pltpu.VMEM((RPW, EMB), jnp.float32),
            pltpu.SemaphoreType.DMA,
        ],
    )
    def k(u_hbm, p_hbm, c_hbm, k_hbm, uW, pW, cW, wuW, wcW,
          pu_o, pi_o, pc_o, wu_o, wc_o,
          iu, ip, ic, ik, buf, sem):
        wid = lax.axis_index("s") * NC + lax.axis_index("c")
        base = wid * RPW
        pltpu.sync_copy(u_hbm.at[wid], iu)
        pltpu.sync_copy(p_hbm.at[wid], ip)
        pltpu.sync_copy(c_hbm.at[wid], ic)
        pltpu.sync_copy(k_hbm.at[wid], ik)
        for table, idx, out in (
            (uW, iu, pu_o),
            (pW, ip, pi_o),
            (wuW, iu, wu_o),
            (wcW, ic, wc_o),
            (cW, ik, pc_o),
        ):
            cps = [
                pltpu.async_copy(table.at[idx.at[j]], buf.at[pl.ds(j * SUB, SUB)], sem)
                for j in range(NSUB)
            ]
            for cp in cps:
                cp.wait()
            pltpu.sync_copy(buf, out.at[pl.ds(base, RPW)])

    return k(u3, p3, c3, k3, user_W, product_W, cluster_W, wide_user_W, wide_cat_W)


def _tc_body(pu_r, pi_r, pc_r, wu_r, wc_r, bs_r,
             A1_r, A2_r, A3_r, w1v_r, b1_r, A4_r, b2_r, A5_r, b3_r, A6_r, b4_r,
             o_r):
    f32 = jnp.float32
    bf16 = jnp.bfloat16
    acc = jnp.dot(pu_r[...].astype(bf16), A1_r[...], preferred_element_type=f32)
    acc = acc + jnp.dot(pi_r[...].astype(bf16), A2_r[...], preferred_element_type=f32)
    acc = acc + jnp.dot(pc_r[...].astype(bf16), A3_r[...], preferred_element_type=f32)
    acc = acc + bs_r[...] * w1v_r[...] + b1_r[...]
    h = jnp.maximum(acc, 0.0).astype(bf16)
    h = jnp.maximum(jnp.dot(h, A4_r[...], preferred_element_type=f32) + b2_r[...], 0.0).astype(bf16)
    h = jnp.maximum(jnp.dot(h, A5_r[...], preferred_element_type=f32) + b3_r[...], 0.0).astype(bf16)
    logit = jnp.dot(h, A6_r[...], preferred_element_type=f32) + b4_r[...]
    wide = jnp.sum(wu_r[...] * wc_r[...], axis=1, keepdims=True)
    o_r[...] = 1.0 / (1.0 + jnp.exp(-(logit + wide)))


def _tc_mlp(pu, pi, pc, wu, wc, bs2,
            A1, A2, A3, w1v, b1s, A4, b2s, A5, b3s, A6, b4r):
    bm = 512
    grid = (B // bm,)

    def blk(shape):
        return pl.BlockSpec(shape, lambda i: (i, 0))

    def full(a):
        return pl.BlockSpec(a.shape, lambda i: (0, 0))

    return pl.pallas_call(
        _tc_body,
        grid=grid,
        in_specs=[
            blk((bm, EMB)), blk((bm, EMB)), blk((bm, EMB)),
            blk((bm, EMB)), blk((bm, EMB)), blk((bm, 1)),
            full(A1), full(A2), full(A3), full(w1v), full(b1s),
            full(A4), full(b2s), full(A5), full(b3s), full(A6), full(b4r),
        ],
        out_specs=blk((bm, 1)),
        out_shape=jax.ShapeDtypeStruct((B, 1), jnp.float32),
    )(pu, pi, pc, wu, wc, bs2, A1, A2, A3, w1v, b1s, A4, b2s, A5, b3s, A6, b4r)


def kernel(user_ids, product_ids, category_ids, cluster_ids, behavior_scores,
           wide_user_W, wide_cat_W, user_W, product_W, cluster_W,
           W1, b1, W2, b2, W3, b3, W4, b4):
    u3 = user_ids.astype(jnp.int32).reshape(NW, NSUB, SUB)
    p3 = product_ids.astype(jnp.int32).reshape(NW, NSUB, SUB)
    c3 = category_ids.astype(jnp.int32).reshape(NW, NSUB, SUB)
    k3 = cluster_ids.astype(jnp.int32).reshape(NW, NSUB, SUB)

    cluster_Wp = jnp.pad(cluster_W, ((0, 0), (0, EMB // 2)))
    pu, pi, pc, wu, wc = _sc_gather(
        u3, p3, c3, k3, user_W, product_W, cluster_Wp, wide_user_W, wide_cat_W)

    s = float(1.0 / np.sqrt(1.0 + 1e-5))
    bf16 = jnp.bfloat16
    W1s = W1 * s
    A1 = W1s[:, :EMB].T.astype(bf16)
    A2 = W1s[:, EMB:2 * EMB].T.astype(bf16)
    A3 = jnp.pad(W1s[:, 2 * EMB:2 * EMB + EMB // 2].T, ((0, EMB // 2), (0, 0))).astype(bf16)
    w1v = W1s[:, 2 * EMB + EMB // 2].reshape(1, -1)
    b1s = (b1 * s).reshape(1, -1)
    A4 = (W2 * s).T.astype(bf16)
    b2s = (b2 * s).reshape(1, -1)
    A5 = (W3 * s).T.astype(bf16)
    b3s = (b3 * s).reshape(1, -1)
    A6 = W4.T.astype(bf16)
    b4r = b4.reshape(1, 1)

    out2 = _tc_mlp(pu, pi, pc, wu, wc, behavior_scores.reshape(B, 1),
                   A1, A2, A3, w1v, b1s, A4, b2s, A5, b3s, A6, b4r)
    return out2.reshape(B)


# speedup vs baseline: 2.9843x; 1.0245x over previous
import functools

import jax
import jax.numpy as jnp
import numpy as np
from jax import lax
from jax.experimental import pallas as pl
from jax.experimental.pallas import tpu as pltpu
from jax.experimental.pallas import tpu_sc as plsc

B = 16384
EMB = 128
FEAT = 3 * EMB
NC = 2
NS = 16
NW = NC * NS
RPW = B // NW
SUB = 128
NSUB = RPW // SUB


def _sc_gather(u3, p3, c3, k3, user_W, product_W, cluster_Wp, wide_user_W, wide_cat_W):
    mesh = plsc.VectorSubcoreMesh(core_axis_name="c", subcore_axis_name="s")

    @functools.partial(
        pl.kernel,
        mesh=mesh,
        out_type=[
            jax.ShapeDtypeStruct((B, FEAT), jnp.float32),
            jax.ShapeDtypeStruct((B, EMB), jnp.float32),
            jax.ShapeDtypeStruct((B, EMB), jnp.float32),
        ],
        scratch_types=[
            pltpu.VMEM((NSUB, SUB), jnp.int32),
            pltpu.VMEM((NSUB, SUB), jnp.int32),
            pltpu.VMEM((NSUB, SUB), jnp.int32),
            pltpu.VMEM((NSUB, SUB), jnp.int32),
            pltpu.VMEM((RPW, EMB), jnp.float32),
            pltpu.SemaphoreType.DMA,
        ],
    )
    def k(u_hbm, p_hbm, c_hbm, k_hbm, uW, pW, cW, wuW, wcW,
          feat_o, wu_o, wc_o,
          iu, ip, ic, ik, buf, sem):
        wid = lax.axis_index("s") * NC + lax.axis_index("c")
        base = wid * RPW
        pltpu.sync_copy(u_hbm.at[wid], iu)
        pltpu.sync_copy(p_hbm.at[wid], ip)
        pltpu.sync_copy(c_hbm.at[wid], ic)
        pltpu.sync_copy(k_hbm.at[wid], ik)
        for table, idx, out_ref in (
            (uW, iu, feat_o.at[pl.ds(base, RPW), pl.ds(0, EMB)]),
            (pW, ip, feat_o.at[pl.ds(base, RPW), pl.ds(EMB, EMB)]),
            (cW, ik, feat_o.at[pl.ds(base, RPW), pl.ds(2 * EMB, EMB)]),
            (wuW, iu, wu_o.at[pl.ds(base, RPW)]),
            (wcW, ic, wc_o.at[pl.ds(base, RPW)]),
        ):
            cps = [
                pltpu.async_copy(table.at[idx.at[j]], buf.at[pl.ds(j * SUB, SUB)], sem)
                for j in range(NSUB)
            ]
            for cp in cps:
                cp.wait()
            pltpu.sync_copy(buf, out_ref)

    return k(u3, p3, c3, k3, user_W, product_W, cluster_Wp, wide_user_W, wide_cat_W)


def _xwt(x, w):
    return lax.dot_general(x, w, (((1,), (1,)), ((), ())),
                           preferred_element_type=jnp.float32)


def _tc_body(feat_r, wu_r, wc_r, bs_r,
             B1_r, w1v_r, b1_r, B2_r, b2_r, B3_r, b3_r, a6_r, b4_r,
             o_r):
    bf16 = jnp.bfloat16
    acc = _xwt(feat_r[...].astype(bf16), B1_r[...])
    acc = acc + bs_r[...] * w1v_r[...] + b1_r[...]
    h = jnp.maximum(acc, 0.0).astype(bf16)
    h = jnp.maximum(_xwt(h, B2_r[...]) + b2_r[...], 0.0).astype(bf16)
    h = jnp.maximum(_xwt(h, B3_r[...]) + b3_r[...], 0.0)
    logit = jnp.sum(h * a6_r[...], axis=1, keepdims=True) + b4_r[...]
    wide = jnp.sum(wu_r[...] * wc_r[...], axis=1, keepdims=True)
    o_r[...] = 1.0 / (1.0 + jnp.exp(-(logit + wide)))


def _tc_mlp(feat, wu, wc, bs2, B1, w1v, b1s, B2, b2s, B3, b3s, a6, b4r):
    bm = 512
    grid = (B // bm,)

    def blk(shape):
        return pl.BlockSpec(shape, lambda i: (i, 0))

    def full(a):
        return pl.BlockSpec(a.shape, lambda i: (0, 0))

    return pl.pallas_call(
        _tc_body,
        grid=grid,
        in_specs=[
            blk((bm, FEAT)), blk((bm, EMB)), blk((bm, EMB)), blk((bm, 1)),
            full(B1), full(w1v), full(b1s),
            full(B2), full(b2s), full(B3), full(b3s), full(a6), full(b4r),
        ],
        out_specs=blk((bm, 1)),
        out_shape=jax.ShapeDtypeStruct((B, 1), jnp.float32),
    )(feat, wu, wc, bs2, B1, w1v, b1s, B2, b2s, B3, b3s, a6, b4r)


def kernel(user_ids, product_ids, category_ids, cluster_ids, behavior_scores,
           wide_user_W, wide_cat_W, user_W, product_W, cluster_W,
           W1, b1, W2, b2, W3, b3, W4, b4):
    u3 = user_ids.astype(jnp.int32).reshape(NW, NSUB, SUB)
    p3 = product_ids.astype(jnp.int32).reshape(NW, NSUB, SUB)
    c3 = category_ids.astype(jnp.int32).reshape(NW, NSUB, SUB)
    k3 = cluster_ids.astype(jnp.int32).reshape(NW, NSUB, SUB)

    cluster_Wp = jnp.pad(cluster_W, ((0, 0), (0, EMB // 2)))
    feat, wu, wc = _sc_gather(
        u3, p3, c3, k3, user_W, product_W, cluster_Wp, wide_user_W, wide_cat_W)

    s = float(1.0 / np.sqrt(1.0 + 1e-5))
    bf16 = jnp.bfloat16
    W1s = W1 * s
    B1 = jnp.pad(W1s[:, :2 * EMB + EMB // 2], ((0, 0), (0, EMB // 2))).astype(bf16)
    w1v = W1s[:, 2 * EMB + EMB // 2].reshape(1, -1)
    b1s = (b1 * s).reshape(1, -1)
    B2 = (W2 * s).astype(bf16)
    b2s = (b2 * s).reshape(1, -1)
    B3 = (W3 * s).astype(bf16)
    b3s = (b3 * s).reshape(1, -1)
    a6 = W4.reshape(1, -1)
    b4r = b4.reshape(1, 1)

    out2 = _tc_mlp(feat, wu, wc, behavior_scores.reshape(B, 1),
                   B1, w1v, b1s, B2, b2s, B3, b3s, a6, b4r)
    return out2.reshape(B)


# speedup vs baseline: 3.1056x; 1.0406x over previous
import functools

import jax
import jax.numpy as jnp
import numpy as np
from jax import lax
from jax.experimental import pallas as pl
from jax.experimental.pallas import tpu as pltpu
from jax.experimental.pallas import tpu_sc as plsc

B = 16384
C = 4
BC = B // C
EMB = 128
FEAT = 3 * EMB
BS_COL = 2 * EMB + EMB // 2
NC = 2
NS = 16
NW = NC * NS
RPW = BC // NW
SUB = 128
NSUB = RPW // SUB


def _sc_gather(u3, p3, c3, k3, bs3, user_W, product_W, cluster_Wp,
               wide_user_W, wide_cat_W):
    mesh = plsc.VectorSubcoreMesh(core_axis_name="c", subcore_axis_name="s")

    @functools.partial(
        pl.kernel,
        mesh=mesh,
        out_type=[
            jax.ShapeDtypeStruct((BC, FEAT), jnp.float32),
            jax.ShapeDtypeStruct((BC, EMB), jnp.float32),
            jax.ShapeDtypeStruct((BC, EMB), jnp.float32),
        ],
        scratch_types=[
            pltpu.VMEM((NSUB, SUB), jnp.int32),
            pltpu.VMEM((NSUB, SUB), jnp.int32),
            pltpu.VMEM((NSUB, SUB), jnp.int32),
            pltpu.VMEM((NSUB, SUB), jnp.int32),
            pltpu.VMEM((NSUB, SUB), jnp.float32),
            pltpu.VMEM((RPW, EMB), jnp.float32),
            pltpu.SemaphoreType.DMA,
        ],
    )
    def k(u_hbm, p_hbm, c_hbm, k_hbm, bs_hbm, uW, pW, cW, wuW, wcW,
          feat_o, wu_o, wc_o,
          iu, ip, ic, ik, bsv, buf, sem):
        wid = lax.axis_index("s") * NC + lax.axis_index("c")
        base = wid * RPW
        pltpu.sync_copy(u_hbm.at[wid], iu)
        pltpu.sync_copy(p_hbm.at[wid], ip)
        pltpu.sync_copy(c_hbm.at[wid], ic)
        pltpu.sync_copy(k_hbm.at[wid], ik)
        pltpu.sync_copy(bs_hbm.at[wid], bsv)
        for table, idx, out_ref, inject_bs in (
            (uW, iu, feat_o.at[pl.ds(base, RPW), pl.ds(0, EMB)], False),
            (pW, ip, feat_o.at[pl.ds(base, RPW), pl.ds(EMB, EMB)], False),
            (wuW, iu, wu_o.at[pl.ds(base, RPW)], False),
            (wcW, ic, wc_o.at[pl.ds(base, RPW)], False),
            (cW, ik, feat_o.at[pl.ds(base, RPW), pl.ds(2 * EMB, EMB)], True),
        ):
            cps = [
                pltpu.async_copy(table.at[idx.at[j]], buf.at[pl.ds(j * SUB, SUB)], sem)
                for j in range(NSUB)
            ]
            for cp in cps:
                cp.wait()
            if inject_bs:
                for j in range(NSUB):
                    for o in range(0, SUB, 16):
                        vals = bsv[j, pl.ds(o, 16)]
                        for l in range(16):
                            v = jnp.full((16,), vals[l], jnp.float32)
                            buf[j * SUB + o + l, pl.ds(EMB // 2, 16)] = v
            pltpu.sync_copy(buf, out_ref)

    return k(u3, p3, c3, k3, bs3, user_W, product_W, cluster_Wp,
             wide_user_W, wide_cat_W)


def _xwt(x, w):
    return lax.dot_general(x, w, (((1,), (1,)), ((), ())),
                           preferred_element_type=jnp.float32)


def _tc_body(feat_r, wu_r, wc_r,
             B1_r, b1_r, B2_r, b2_r, B3_r, b3_r, a6_r, b4_r,
             o_r):
    bf16 = jnp.bfloat16
    acc = _xwt(feat_r[...].astype(bf16), B1_r[...]) + b1_r[...]
    h = jnp.maximum(acc, 0.0).astype(bf16)
    h = jnp.maximum(_xwt(h, B2_r[...]) + b2_r[...], 0.0).astype(bf16)
    h = jnp.maximum(_xwt(h, B3_r[...]) + b3_r[...], 0.0)
    logit = jnp.sum(h * a6_r[...], axis=1) + b4_r[0, 0]
    wide = jnp.sum(wu_r[...] * wc_r[...], axis=1)
    o_r[...] = 1.0 / (1.0 + jnp.exp(-(logit + wide)))


def _tc_mlp(feat, wu, wc, B1, b1s, B2, b2s, B3, b3s, a6, b4r):
    bm = 512
    grid = (BC // bm,)

    def blk(shape):
        return pl.BlockSpec(shape, lambda i: (i, 0))

    def full(a):
        return pl.BlockSpec(a.shape, lambda i: (0,) * a.ndim)

    return pl.pallas_call(
        _tc_body,
        grid=grid,
        in_specs=[
            blk((bm, FEAT)), blk((bm, EMB)), blk((bm, EMB)),
            full(B1), full(b1s), full(B2), full(b2s), full(B3), full(b3s),
            full(a6), full(b4r),
        ],
        out_specs=pl.BlockSpec((bm,), lambda i: (i,)),
        out_shape=jax.ShapeDtypeStruct((BC,), jnp.float32),
    )(feat, wu, wc, B1, b1s, B2, b2s, B3, b3s, a6, b4r)


def kernel(user_ids, product_ids, category_ids, cluster_ids, behavior_scores,
           wide_user_W, wide_cat_W, user_W, product_W, cluster_W,
           W1, b1, W2, b2, W3, b3, W4, b4):
    u4 = user_ids.astype(jnp.int32).reshape(C, NW, NSUB, SUB)
    p4 = product_ids.astype(jnp.int32).reshape(C, NW, NSUB, SUB)
    c4 = category_ids.astype(jnp.int32).reshape(C, NW, NSUB, SUB)
    k4 = cluster_ids.astype(jnp.int32).reshape(C, NW, NSUB, SUB)
    bs4 = behavior_scores.reshape(C, NW, NSUB, SUB)

    cluster_Wp = jnp.pad(cluster_W, ((0, 0), (0, EMB // 2)))

    s = float(1.0 / np.sqrt(1.0 + 1e-5))
    bf16 = jnp.bfloat16
    B1 = jnp.pad(W1 * s, ((0, 0), (0, FEAT - W1.shape[1]))).astype(bf16)
    b1s = (b1 * s).reshape(1, -1)
    B2 = (W2 * s).astype(bf16)
    b2s = (b2 * s).reshape(1, -1)
    B3 = (W3 * s).astype(bf16)
    b3s = (b3 * s).reshape(1, -1)
    a6 = W4.reshape(1, -1)
    b4r = b4.reshape(1, 1)

    outs = []
    for c in range(C):
        feat, wu, wc = _sc_gather(
            u4[c], p4[c], c4[c], k4[c], bs4[c],
            user_W, product_W, cluster_Wp, wide_user_W, wide_cat_W)
        outs.append(_tc_mlp(feat, wu, wc, B1, b1s, B2, b2s, B3, b3s, a6, b4r))
    return jnp.concatenate(outs)
